# baseline (speedup 1.0000x reference)
import functools

import jax
import jax.numpy as jnp
from jax import lax
from jax.experimental import pallas as pl
from jax.experimental.pallas import tpu as pltpu
from jax.experimental.pallas import tpu_sc as plsc

VOCAB = 1000000
EMBED_DIM = 4
BATCH = 16384
LINE = 16
PLANE_LINES = VOCAB // LINE
N_LINES = PLANE_LINES * EMBED_DIM
BLOCK = 128

_NUM_CORES = 2
_NUM_SUBCORES = 16
_NUM_WORKERS = _NUM_CORES * _NUM_SUBCORES
_B_PER_W = BATCH // _NUM_WORKERS
_CHUNK = 128
_E_PER_W = _B_PER_W * EMBED_DIM
_N_GATHERS = _E_PER_W // _CHUNK
_LANES = 16

_mesh = plsc.VectorSubcoreMesh(core_axis_name="c", subcore_axis_name="s")


@functools.partial(
    pl.kernel,
    mesh=_mesh,
    compiler_params=pltpu.CompilerParams(
        use_tc_tiling_on_sc=False, needs_layout_passes=False
    ),
    out_type=jax.ShapeDtypeStruct((BATCH * EMBED_DIM,), jnp.float32),
    scratch_types=[
        pltpu.VMEM((_B_PER_W,), jnp.int32),
        pltpu.VMEM((_N_GATHERS, _CHUNK), jnp.int32),
        pltpu.VMEM((_E_PER_W, LINE), jnp.float32),
        pltpu.VMEM((_E_PER_W,), jnp.float32),
        pltpu.SemaphoreType.DMA,
    ],
)
def _embed_gather(e_hbm, lines_hbm, out_hbm, idx_v, addr_v, lines_v, outb_v, sem):
    wid = lax.axis_index("s") * _NUM_CORES + lax.axis_index("c")
    base = wid * _B_PER_W
    pltpu.sync_copy(e_hbm.at[pl.ds(base, _B_PER_W)], idx_v)

    lane = lax.iota(jnp.int32, _LANES)
    copies = []
    for j in range(_N_GATHERS):
        c, g = j >> 2, j & 3
        for u in range(_CHUNK // _LANES):
            lb = idx_v[pl.ds(g * _CHUNK + u * _LANES, _LANES)] >> 4
            addr_v[j, pl.ds(u * _LANES, _LANES)] = lb + c * PLANE_LINES
        copies.append(
            pltpu.async_copy(
                lines_hbm.at[addr_v.at[j]],
                lines_v.at[pl.ds(j * _CHUNK, _CHUNK)],
                sem,
            )
        )
    for cp in copies:
        cp.wait()

    for i in range(_E_PER_W // _LANES):
        c = (i >> 3) & 3
        k = lane + 16 * (i & 7) + 128 * (i >> 5)
        r = plsc.load_gather(idx_v, [k])
        vals = plsc.load_gather(lines_v, [k + c * _B_PER_W, r & 15])
        outb_v[pl.ds(i * _LANES, _LANES)] = vals

    pltpu.sync_copy(outb_v, out_hbm.at[pl.ds(base * EMBED_DIM, _E_PER_W)])


def kernel(e, table):
    lines = lax.reshape(table, (N_LINES, LINE), dimensions=(1, 0))
    out_flat = _embed_gather(e.astype(jnp.int32), lines)
    out = jnp.transpose(
        jnp.reshape(out_flat, (BATCH // BLOCK, EMBED_DIM, BLOCK)), (0, 2, 1)
    ).reshape(BATCH, EMBED_DIM)
    return out


# speedup vs baseline: 1.0207x; 1.0207x over previous
import functools

import jax
import jax.numpy as jnp
from jax import lax
from jax.experimental import pallas as pl
from jax.experimental.pallas import tpu as pltpu
from jax.experimental.pallas import tpu_sc as plsc

VOCAB = 1000000
EMBED_DIM = 4
BATCH = 16384
LINE = 16
PLANE_LINES = VOCAB // LINE
N_LINES = PLANE_LINES * EMBED_DIM
BLOCK = 128

_NUM_CORES = 2
_NUM_SUBCORES = 16
_NUM_WORKERS = _NUM_CORES * _NUM_SUBCORES
_B_PER_W = BATCH // _NUM_WORKERS
_CHUNK = 128
_E_PER_W = _B_PER_W * EMBED_DIM
_N_GATHERS = _E_PER_W // _CHUNK
_LANES = 16

_mesh = plsc.VectorSubcoreMesh(core_axis_name="c", subcore_axis_name="s")


@functools.partial(
    pl.kernel,
    mesh=_mesh,
    compiler_params=pltpu.CompilerParams(
        use_tc_tiling_on_sc=False, needs_layout_passes=False
    ),
    out_type=jax.ShapeDtypeStruct((BATCH * EMBED_DIM,), jnp.float32),
    scratch_types=[
        pltpu.VMEM((_B_PER_W,), jnp.int32),
        pltpu.VMEM((_N_GATHERS, _CHUNK), jnp.int32),
        pltpu.VMEM((_E_PER_W, LINE), jnp.float32),
        pltpu.VMEM((_E_PER_W,), jnp.float32),
        pltpu.SemaphoreType.DMA,
    ],
)
def _embed_gather(e_hbm, lines_hbm, out_hbm, idx_v, addr_v, lines_v, outb_v, sem):
    wid = lax.axis_index("s") * _NUM_CORES + lax.axis_index("c")
    base = wid * _B_PER_W
    pltpu.sync_copy(e_hbm.at[pl.ds(base, _B_PER_W)], idx_v)

    lane = lax.iota(jnp.int32, _LANES)
    copies = []
    for j in range(_N_GATHERS):
        c, g = j >> 2, j & 3
        for u in range(_CHUNK // _LANES):
            lb = idx_v[pl.ds(g * _CHUNK + u * _LANES, _LANES)] >> 4
            addr_v[j, pl.ds(u * _LANES, _LANES)] = lb + c * PLANE_LINES
        copies.append(
            pltpu.async_copy(
                lines_hbm.at[addr_v.at[j]],
                lines_v.at[pl.ds(j * _CHUNK, _CHUNK)],
                sem,
            )
        )

    for j in range(_N_GATHERS):
        copies[j].wait()
        c, g = j >> 2, j & 3
        for t in range(_CHUNK // _LANES):
            i = 32 * g + 8 * c + t
            k = lane + 16 * t + 128 * g
            r = plsc.load_gather(idx_v, [k])
            vals = plsc.load_gather(lines_v, [k + c * _B_PER_W, r & 15])
            outb_v[pl.ds(i * _LANES, _LANES)] = vals

    pltpu.sync_copy(outb_v, out_hbm.at[pl.ds(base * EMBED_DIM, _E_PER_W)])


def kernel(e, table):
    lines = lax.reshape(table, (N_LINES, LINE), dimensions=(1, 0))
    out_flat = _embed_gather(e.astype(jnp.int32), lines)
    out = jnp.transpose(
        jnp.reshape(out_flat, (BATCH // BLOCK, EMBED_DIM, BLOCK)), (0, 2, 1)
    ).reshape(BATCH, EMBED_DIM)
    return out


# speedup vs baseline: 1.0287x; 1.0078x over previous
import functools

import jax
import jax.numpy as jnp
from jax import lax
from jax.experimental import pallas as pl
from jax.experimental.pallas import tpu as pltpu
from jax.experimental.pallas import tpu_sc as plsc

VOCAB = 1000000
EMBED_DIM = 4
BATCH = 16384
LINE = 16
PLANE_LINES = VOCAB // LINE
N_LINES = PLANE_LINES * EMBED_DIM
BLOCK = 128

_NUM_CORES = 2
_NUM_SUBCORES = 16
_NUM_WORKERS = _NUM_CORES * _NUM_SUBCORES
_B_PER_W = BATCH // _NUM_WORKERS
_CHUNK = 128
_E_PER_W = _B_PER_W * EMBED_DIM
_N_GATHERS = _E_PER_W // _CHUNK
_LANES = 16

_mesh = plsc.VectorSubcoreMesh(core_axis_name="c", subcore_axis_name="s")


@functools.partial(
    pl.kernel,
    mesh=_mesh,
    compiler_params=pltpu.CompilerParams(
        use_tc_tiling_on_sc=False, needs_layout_passes=False
    ),
    out_type=jax.ShapeDtypeStruct((BATCH * EMBED_DIM,), jnp.float32),
    scratch_types=[
        pltpu.VMEM((_B_PER_W,), jnp.int32),
        pltpu.VMEM((_N_GATHERS, _CHUNK), jnp.int32),
        pltpu.VMEM((_E_PER_W, LINE), jnp.float32),
        pltpu.VMEM((_E_PER_W,), jnp.float32),
        pltpu.SemaphoreType.DMA,
    ],
)
def _embed_gather(e_hbm, lines_hbm, out_hbm, idx_v, addr_v, lines_v, outb_v, sem):
    wid = lax.axis_index("s") * _NUM_CORES + lax.axis_index("c")
    base = wid * _B_PER_W
    pltpu.sync_copy(e_hbm.at[pl.ds(base, _B_PER_W)], idx_v)

    lane = lax.iota(jnp.int32, _LANES)
    copies = []
    for j in range(_N_GATHERS):
        c, g = j >> 2, j & 3
        for u in range(_CHUNK // _LANES):
            lb = idx_v[pl.ds(g * _CHUNK + u * _LANES, _LANES)] >> 4
            addr_v[j, pl.ds(u * _LANES, _LANES)] = lb + c * PLANE_LINES
        copies.append(
            pltpu.async_copy(
                lines_hbm.at[addr_v.at[j]],
                lines_v.at[pl.ds(j * _CHUNK, _CHUNK)],
                sem,
            )
        )

    for j in range(_N_GATHERS):
        copies[j].wait()
        c, g = j >> 2, j & 3
        for t in range(_CHUNK // _LANES):
            i = 32 * g + 8 * c + t
            k = lane + 16 * t + 128 * g
            r = idx_v[pl.ds(16 * t + 128 * g, _LANES)]
            vals = plsc.load_gather(lines_v, [k + c * _B_PER_W, r & 15])
            outb_v[pl.ds(i * _LANES, _LANES)] = vals

    pltpu.sync_copy(outb_v, out_hbm.at[pl.ds(base * EMBED_DIM, _E_PER_W)])


def kernel(e, table):
    lines = lax.reshape(table, (N_LINES, LINE), dimensions=(1, 0))
    out_flat = _embed_gather(e.astype(jnp.int32), lines)
    out = jnp.transpose(
        jnp.reshape(out_flat, (BATCH // BLOCK, EMBED_DIM, BLOCK)), (0, 2, 1)
    ).reshape(BATCH, EMBED_DIM)
    return out
